# baseline (speedup 1.0000x reference)
import functools

import jax
import jax.numpy as jnp
from jax import lax
from jax.experimental import pallas as pl
from jax.experimental.pallas import tpu as pltpu
from jax.experimental.pallas import tpu_sc as plsc

N, M, FIN, E, KD, F1 = 4, 10000, 128, 320000, 3, 128
MP = 10240
C = FIN
NC, NS, L = 2, 16, 16
EPT = E // NS
B = 80
NB = EPT // B
RPT = MP // NS
RB = 64
NRB = RPT // RB
NQ = C // L


G = 10
NBG = NB // G


def _scale_rows(rows, valv, jj):
  def grp(g, carry):
    vals = valv[jj, pl.ds(g * L, L)]
    for u in range(L):
      e = g * L + u
      bc = jnp.full((L,), vals[u], jnp.float32)
      for q in range(NQ):
        sl = pl.ds(q * L, L)
        rows[e, sl] = rows[e, sl] * bc
    return carry

  lax.fori_loop(0, B // L, grp, 0)


def _sc_body(x_hbm, row_hbm, col_hbm, val_hbm, x1_hbm, s1_hbm,
             acc, colv, rowv, valv, rows_a, rows_b, fbs, fbx, sem_a, sem_b,
             ssem_a, ssem_b):
  cid = lax.axis_index("c")
  sid = lax.axis_index("s")
  r0 = sid * RPT

  def zero_acc():
    def zr(r, carry):
      for q in range(NQ):
        fbs[r, pl.ds(q * L, L)] = jnp.zeros((L,), jnp.float32)
      return carry

    lax.fori_loop(0, RB, zr, 0)
    for i in range(NRB):
      pltpu.sync_copy(fbs, acc.at[pl.ds(r0 + i * RB, RB)])

  def scatter_pass(table_hbm, chunk):
    zero_acc()
    plsc.subcore_barrier()
    tab = table_hbm.at[chunk]

    def gwait(buf, sem):
      pltpu.make_async_copy(tab.at[pl.ds(0, B)], buf, sem).wait()

    def swait(buf, sem):
      pltpu.make_async_copy(buf, acc.at[pl.ds(0, B)], sem).wait()

    def block(jg, carry):
      pltpu.sync_copy(col_hbm.at[sid, jg], colv)
      pltpu.sync_copy(row_hbm.at[sid, jg], rowv)
      pltpu.sync_copy(val_hbm.at[sid, jg], valv)
      pltpu.async_copy(tab.at[colv.at[0]], rows_a, sem_a)

      def pair(jp, c2):
        j0 = 2 * jp
        gwait(rows_a, sem_a)

        @pl.when((jp > 0) | (jg > 0))
        def _():
          swait(rows_b, ssem_b)

        pltpu.async_copy(tab.at[colv.at[j0 + 1]], rows_b, sem_b)
        _scale_rows(rows_a, valv, j0)
        pltpu.async_copy(rows_a, acc.at[rowv.at[j0]], ssem_a, add=True)
        gwait(rows_b, sem_b)
        swait(rows_a, ssem_a)

        @pl.when(jp < G // 2 - 1)
        def _():
          pltpu.async_copy(tab.at[colv.at[j0 + 2]], rows_a, sem_a)

        _scale_rows(rows_b, valv, j0 + 1)
        pltpu.async_copy(rows_b, acc.at[rowv.at[j0 + 1]], ssem_b, add=True)
        return c2

      lax.fori_loop(0, G // 2, pair, 0)
      return carry

    lax.fori_loop(0, NBG, block, 0)
    swait(rows_b, ssem_b)
    plsc.subcore_barrier()

  for i in range(2):
    chunk = cid * 2 + i
    scatter_pass(x_hbm, chunk)
    for blk in range(NRB):
      rr = r0 + blk * RB
      pltpu.sync_copy(acc.at[pl.ds(rr, RB)], fbs)
      pltpu.sync_copy(x_hbm.at[chunk].at[pl.ds(rr, RB)], fbx)

      def sub(r, carry):
        for q in range(NQ):
          sl = pl.ds(q * L, L)
          fbs[r, sl] = fbs[r, sl] - fbx[r, sl]
        return carry

      lax.fori_loop(0, RB, sub, 0)
      pltpu.sync_copy(fbs, x1_hbm.at[chunk].at[pl.ds(rr, RB)])
    plsc.subcore_barrier()

    scatter_pass(x1_hbm, chunk)
    for blk in range(NRB):
      rr = r0 + blk * RB
      pltpu.sync_copy(acc.at[pl.ds(rr, RB)], fbs)
      pltpu.sync_copy(fbs, s1_hbm.at[chunk].at[pl.ds(rr, RB)])
    plsc.subcore_barrier()


_sc_sparse = functools.partial(
    pl.kernel,
    out_type=(
        jax.ShapeDtypeStruct((N, MP, C), jnp.float32),
        jax.ShapeDtypeStruct((N, MP, C), jnp.float32),
    ),
    mesh=plsc.VectorSubcoreMesh(
        core_axis_name="c", subcore_axis_name="s", num_cores=NC,
        num_subcores=NS),
    compiler_params=pltpu.CompilerParams(use_tc_tiling_on_sc=False),
    scratch_types=[
        pltpu.VMEM_SHARED((MP, C), jnp.float32),
        pltpu.VMEM((G, B), jnp.int32),
        pltpu.VMEM((G, B), jnp.int32),
        pltpu.VMEM((G, B), jnp.float32),
        pltpu.VMEM((B, C), jnp.float32),
        pltpu.VMEM((B, C), jnp.float32),
        pltpu.VMEM((RB, C), jnp.float32),
        pltpu.VMEM((RB, C), jnp.float32),
        pltpu.SemaphoreType.DMA,
        pltpu.SemaphoreType.DMA,
        pltpu.SemaphoreType.DMA,
        pltpu.SemaphoreType.DMA,
    ],
)(_sc_body)


BM = 2000


def _mm_body(x_ref, x1_ref, s1_ref, wa_ref, wb_ref, wc_ref, bias_ref, o_ref):
  acc = jnp.dot(x_ref[0], wa_ref[...], preferred_element_type=jnp.float32,
                precision=lax.Precision.HIGHEST)
  acc += jnp.dot(x1_ref[0], wb_ref[...], preferred_element_type=jnp.float32,
                 precision=lax.Precision.HIGHEST)
  acc += jnp.dot(s1_ref[0], wc_ref[...], preferred_element_type=jnp.float32,
                 precision=lax.Precision.HIGHEST)
  o_ref[0] = jnp.maximum(acc + bias_ref[0, 0][None, :], 0.0)


def _tc_matmul(x, x1, s1, wa, wb, wc, bias):
  grid = (N, M // BM)
  blk = lambda n, m: (n, m, 0)
  zero3 = lambda n, m: (0, 0, 0)
  return pl.pallas_call(
      _mm_body,
      grid=grid,
      in_specs=[
          pl.BlockSpec((1, BM, FIN), blk),
          pl.BlockSpec((1, BM, FIN), blk),
          pl.BlockSpec((1, BM, FIN), blk),
          pl.BlockSpec((FIN, F1), lambda n, m: (0, 0)),
          pl.BlockSpec((FIN, F1), lambda n, m: (0, 0)),
          pl.BlockSpec((FIN, F1), lambda n, m: (0, 0)),
          pl.BlockSpec((1, 1, F1), zero3),
      ],
      out_specs=pl.BlockSpec((1, BM, F1), blk),
      out_shape=jax.ShapeDtypeStruct((N, M, F1), jnp.float32),
  )(x, x1, s1, wa, wb, wc, bias)


@jax.jit
def kernel(x, edge_row, edge_col, edge_val, kernel, bias):
  xp = jnp.pad(x, ((0, 0), (0, MP - M), (0, 0)))
  row4 = edge_row.reshape(NS, NBG, G, B)
  col4 = edge_col.reshape(NS, NBG, G, B)
  val4 = edge_val.reshape(NS, NBG, G, B)
  x1p, s1p = _sc_sparse(xp, row4, col4, val4)
  x1 = x1p[:, :M, :]
  s1 = s1p[:, :M, :]
  w3 = kernel.reshape(FIN, KD, F1)
  wa = w3[:, 0, :] - w3[:, 2, :]
  wb = w3[:, 1, :] - 2.0 * w3[:, 2, :]
  wc = 2.0 * w3[:, 2, :]
  return _tc_matmul(x, x1, s1, wa, wb, wc, bias)


# speedup vs baseline: 1.1434x; 1.1434x over previous
import functools

import jax
import jax.numpy as jnp
from jax import lax
from jax.experimental import pallas as pl
from jax.experimental.pallas import tpu as pltpu
from jax.experimental.pallas import tpu_sc as plsc

N, M, FIN, E, KD, F1 = 4, 10000, 128, 320000, 3, 128
MP = 10240
C = FIN
NC, NS, L = 2, 16, 16
EPT = E // NS
B = 80
NB = EPT // B
RPT = MP // NS
RB = 64
NRB = RPT // RB
NQ = C // L


G = 10
NBG = NB // G


def _scale_rows(rows, valv, slot, jj):
  def grp(g, carry):
    vals = valv[slot, jj, pl.ds(g * L, L)]
    for u in range(L):
      e = g * L + u
      bc = jnp.full((L,), vals[u], jnp.float32)
      for q in range(NQ):
        sl = pl.ds(q * L, L)
        rows[e, sl] = rows[e, sl] * bc
    return carry

  lax.fori_loop(0, B // L, grp, 0)


def _sc_body(x_hbm, row_hbm, col_hbm, val_hbm, x1_hbm, s1_hbm,
             acc, colv, rowv, valv, rows_a, rows_b, fbs, fbx, sem_a, sem_b,
             ssem_a, ssem_b, isem):
  cid = lax.axis_index("c")
  sid = lax.axis_index("s")
  r0 = sid * RPT

  def zero_acc():
    def zr(r, carry):
      for q in range(NQ):
        fbs[r, pl.ds(q * L, L)] = jnp.zeros((L,), jnp.float32)
      return carry

    lax.fori_loop(0, RB, zr, 0)
    for i in range(NRB):
      pltpu.sync_copy(fbs, acc.at[pl.ds(r0 + i * RB, RB)])

  def scatter_pass(table_hbm, chunk):
    zero_acc()
    plsc.subcore_barrier()
    tab = table_hbm.at[chunk]

    def gwait(buf, sem):
      pltpu.make_async_copy(tab.at[pl.ds(0, B)], buf, sem).wait()

    def swait(buf, sem):
      pltpu.make_async_copy(buf, acc.at[pl.ds(0, B)], sem).wait()

    def ifetch(jg, slot):
      pltpu.async_copy(col_hbm.at[sid, jg], colv.at[slot], isem)
      pltpu.async_copy(row_hbm.at[sid, jg], rowv.at[slot], isem)
      pltpu.async_copy(val_hbm.at[sid, jg], valv.at[slot], isem)

    def iwait(slot):
      pltpu.make_async_copy(col_hbm.at[sid, 0], colv.at[slot], isem).wait()
      pltpu.make_async_copy(row_hbm.at[sid, 0], rowv.at[slot], isem).wait()
      pltpu.make_async_copy(val_hbm.at[sid, 0], valv.at[slot], isem).wait()

    pltpu.sync_copy(col_hbm.at[sid, 0], colv.at[0])
    pltpu.sync_copy(row_hbm.at[sid, 0], rowv.at[0])
    pltpu.sync_copy(val_hbm.at[sid, 0], valv.at[0])
    pltpu.async_copy(tab.at[colv.at[0, 0]], rows_a, sem_a)

    def block(jg, carry):
      slot = lax.rem(jg, 2)
      nxt = 1 - slot

      @pl.when(jg > 0)
      def _():
        swait(rows_b, ssem_b)

      @pl.when(jg + 1 < NBG)
      def _():
        ifetch(jg + 1, nxt)

      def pair(jp, c2):
        j0 = 2 * jp
        gwait(rows_a, sem_a)

        @pl.when(jp > 0)
        def _():
          swait(rows_b, ssem_b)

        pltpu.async_copy(tab.at[colv.at[slot, j0 + 1]], rows_b, sem_b)
        _scale_rows(rows_a, valv, slot, j0)
        pltpu.async_copy(rows_a, acc.at[rowv.at[slot, j0]], ssem_a, add=True)
        gwait(rows_b, sem_b)
        swait(rows_a, ssem_a)

        @pl.when(jp < G // 2 - 1)
        def _():
          pltpu.async_copy(tab.at[colv.at[slot, j0 + 2]], rows_a, sem_a)

        @pl.when((jp == G // 2 - 1) & (jg + 1 < NBG))
        def _():
          iwait(nxt)
          pltpu.async_copy(tab.at[colv.at[nxt, 0]], rows_a, sem_a)

        _scale_rows(rows_b, valv, slot, j0 + 1)
        pltpu.async_copy(rows_b, acc.at[rowv.at[slot, j0 + 1]], ssem_b, add=True)
        return c2

      lax.fori_loop(0, G // 2, pair, 0)
      return carry

    lax.fori_loop(0, NBG, block, 0)
    swait(rows_b, ssem_b)
    plsc.subcore_barrier()

  for i in range(2):
    chunk = cid * 2 + i
    scatter_pass(x_hbm, chunk)
    for blk in range(NRB):
      rr = r0 + blk * RB
      pltpu.sync_copy(acc.at[pl.ds(rr, RB)], fbs)
      pltpu.sync_copy(x_hbm.at[chunk].at[pl.ds(rr, RB)], fbx)

      def sub(r, carry):
        for q in range(NQ):
          sl = pl.ds(q * L, L)
          fbs[r, sl] = fbs[r, sl] - fbx[r, sl]
        return carry

      lax.fori_loop(0, RB, sub, 0)
      pltpu.sync_copy(fbs, x1_hbm.at[chunk].at[pl.ds(rr, RB)])
    plsc.subcore_barrier()

    scatter_pass(x1_hbm, chunk)
    for blk in range(NRB):
      rr = r0 + blk * RB
      pltpu.sync_copy(acc.at[pl.ds(rr, RB)], fbs)
      pltpu.sync_copy(fbs, s1_hbm.at[chunk].at[pl.ds(rr, RB)])
    plsc.subcore_barrier()


_sc_sparse = functools.partial(
    pl.kernel,
    out_type=(
        jax.ShapeDtypeStruct((N, MP, C), jnp.float32),
        jax.ShapeDtypeStruct((N, MP, C), jnp.float32),
    ),
    mesh=plsc.VectorSubcoreMesh(
        core_axis_name="c", subcore_axis_name="s", num_cores=NC,
        num_subcores=NS),
    compiler_params=pltpu.CompilerParams(use_tc_tiling_on_sc=False),
    scratch_types=[
        pltpu.VMEM_SHARED((MP, C), jnp.float32),
        pltpu.VMEM((2, G, B), jnp.int32),
        pltpu.VMEM((2, G, B), jnp.int32),
        pltpu.VMEM((2, G, B), jnp.float32),
        pltpu.VMEM((B, C), jnp.float32),
        pltpu.VMEM((B, C), jnp.float32),
        pltpu.VMEM((RB, C), jnp.float32),
        pltpu.VMEM((RB, C), jnp.float32),
        pltpu.SemaphoreType.DMA,
        pltpu.SemaphoreType.DMA,
        pltpu.SemaphoreType.DMA,
        pltpu.SemaphoreType.DMA,
        pltpu.SemaphoreType.DMA,
    ],
)(_sc_body)


BM = 2000


def _mm_body(x_ref, x1_ref, s1_ref, wa_ref, wb_ref, wc_ref, bias_ref, o_ref):
  acc = jnp.dot(x_ref[0], wa_ref[...], preferred_element_type=jnp.float32,
                precision=lax.Precision.HIGHEST)
  acc += jnp.dot(x1_ref[0], wb_ref[...], preferred_element_type=jnp.float32,
                 precision=lax.Precision.HIGHEST)
  acc += jnp.dot(s1_ref[0], wc_ref[...], preferred_element_type=jnp.float32,
                 precision=lax.Precision.HIGHEST)
  o_ref[0] = jnp.maximum(acc + bias_ref[0, 0][None, :], 0.0)


def _tc_matmul(x, x1, s1, wa, wb, wc, bias):
  grid = (N, M // BM)
  blk = lambda n, m: (n, m, 0)
  zero3 = lambda n, m: (0, 0, 0)
  return pl.pallas_call(
      _mm_body,
      grid=grid,
      in_specs=[
          pl.BlockSpec((1, BM, FIN), blk),
          pl.BlockSpec((1, BM, FIN), blk),
          pl.BlockSpec((1, BM, FIN), blk),
          pl.BlockSpec((FIN, F1), lambda n, m: (0, 0)),
          pl.BlockSpec((FIN, F1), lambda n, m: (0, 0)),
          pl.BlockSpec((FIN, F1), lambda n, m: (0, 0)),
          pl.BlockSpec((1, 1, F1), zero3),
      ],
      out_specs=pl.BlockSpec((1, BM, F1), blk),
      out_shape=jax.ShapeDtypeStruct((N, M, F1), jnp.float32),
  )(x, x1, s1, wa, wb, wc, bias)


@jax.jit
def kernel(x, edge_row, edge_col, edge_val, kernel, bias):
  xp = jnp.pad(x, ((0, 0), (0, MP - M), (0, 0)))
  row4 = edge_row.reshape(NS, NBG, G, B)
  col4 = edge_col.reshape(NS, NBG, G, B)
  val4 = edge_val.reshape(NS, NBG, G, B)
  x1p, s1p = _sc_sparse(xp, row4, col4, val4)
  x1 = x1p[:, :M, :]
  s1 = s1p[:, :M, :]
  w3 = kernel.reshape(FIN, KD, F1)
  wa = w3[:, 0, :] - w3[:, 2, :]
  wb = w3[:, 1, :] - 2.0 * w3[:, 2, :]
  wc = 2.0 * w3[:, 2, :]
  return _tc_matmul(x, x1, s1, wa, wb, wc, bias)
